# Initial kernel scaffold; baseline (speedup 1.0000x reference)
import jax
import jax.numpy as jnp
from jax.experimental import pallas as pl


def kernel(x, edge_index, edge_features, params):
    raise NotImplementedError("write your pallas kernel here")


# speedup vs baseline: 5.7831x; 5.7831x over previous
import functools

import jax
import jax.numpy as jnp
from jax import lax
from jax.experimental import pallas as pl
from jax.experimental.pallas import tpu as pltpu
from jax.experimental.pallas import tpu_sc as plsc

NN = 10000
NE = 320000
D = 128

GW = 128
NW = 32
NEP = 327680
NNP = 10240
DUMMY = NN

EBLK = 4096
NBLK = 1000

_SDS = jax.ShapeDtypeStruct
_F32 = jnp.float32
_BF16 = jnp.bfloat16


def _ln(h, g, b):
    mu = jnp.mean(h, axis=-1, keepdims=True)
    var = jnp.mean((h - mu) ** 2, axis=-1, keepdims=True)
    return (h - mu) * lax.rsqrt(var + 1e-5) * g + b


def _dot(a, b):
    return jnp.dot(a, b, preferred_element_type=_F32)


def _full(shape):
    return pl.BlockSpec(shape, lambda i: (0,) * len(shape))


def _enc_node_body(x, w0, b0, w1, b1, w2, b2, g, be, wd, ws,
                   v_o, p_o, q_o):
    h = jnp.maximum(_dot(x[...], w0[...]) + b0[...], 0)
    h = jnp.maximum(_dot(h, w1[...]) + b1[...], 0)
    v = _ln(_dot(h, w2[...]) + b2[...], g[...], be[...])
    v_o[...] = v
    p_o[...] = _dot(v, wd[...])
    q_o[...] = _dot(v, ws[...])


def _enc_node(x, ps, g, be, wd, ws):
    grid = NN // NBLK
    return pl.pallas_call(
        _enc_node_body,
        grid=(grid,),
        in_specs=[pl.BlockSpec((NBLK, 30), lambda i: (i, 0)),
                  _full((30, D)), _full((1, D)),
                  _full((D, D)), _full((1, D)),
                  _full((D, D)), _full((1, D)),
                  _full((1, D)), _full((1, D)),
                  _full((D, D)), _full((D, D))],
        out_specs=[pl.BlockSpec((NBLK, D), lambda i: (i, 0)),
                   pl.BlockSpec((NBLK, D), lambda i: (i, 0)),
                   pl.BlockSpec((NBLK, D), lambda i: (i, 0))],
        out_shape=[_SDS((NN, D), _F32), _SDS((NNP, D), _F32),
                   _SDS((NNP, D), _F32)],
    )(x, ps[0]['w'], ps[0]['b'].reshape(1, D),
      ps[1]['w'], ps[1]['b'].reshape(1, D),
      ps[2]['w'], ps[2]['b'].reshape(1, D),
      g.reshape(1, D), be.reshape(1, D), wd, ws)


def _enc_edge_body(x, w0, b0, w1, b1, w2, b2, g, be, e_o):
    h = jnp.maximum(_dot(x[...], w0[...]) + b0[...], 0)
    h = jnp.maximum(_dot(h, w1[...]) + b1[...], 0)
    e_o[...] = _ln(_dot(h, w2[...]) + b2[...], g[...], be[...]).astype(_BF16)


def _enc_edge(ef, ps, g, be):
    grid = NEP // EBLK
    return pl.pallas_call(
        _enc_edge_body,
        grid=(grid,),
        in_specs=[pl.BlockSpec((EBLK, 3), lambda i: (i, 0)),
                  _full((3, D)), _full((1, D)),
                  _full((D, D)), _full((1, D)),
                  _full((D, D)), _full((1, D)),
                  _full((1, D)), _full((1, D))],
        out_specs=[pl.BlockSpec((EBLK, D), lambda i: (i, 0))],
        out_shape=[_SDS((NEP, D), _BF16)],
    )(ef, ps[0]['w'], ps[0]['b'].reshape(1, D),
      ps[1]['w'], ps[1]['b'].reshape(1, D),
      ps[2]['w'], ps[2]['b'].reshape(1, D),
      g.reshape(1, D), be.reshape(1, D))[0]


def _edge_step_body(a, b, e, we, b0, w1, b1, w2, b2, g, be,
                    eo, mo):
    ev = e[...].astype(_F32)
    h = (a[...].astype(_F32) + b[...].astype(_F32)
         + _dot(ev, we[...]) + b0[...])
    h = jnp.maximum(h, 0)
    h = jnp.maximum(_dot(h, w1[...]) + b1[...], 0)
    m = _ln(_dot(h, w2[...]) + b2[...], g[...], be[...])
    mo[...] = m
    eo[...] = (ev + m).astype(_BF16)


def _edge_step(a, b, e, we, b0, w1, b1, w2, b2, g, be):
    grid = NEP // EBLK
    eb = pl.BlockSpec((EBLK, D), lambda i: (i, 0))
    return pl.pallas_call(
        _edge_step_body,
        grid=(grid,),
        in_specs=[eb, eb, eb,
                  _full((D, D)), _full((1, D)),
                  _full((D, D)), _full((1, D)),
                  _full((D, D)), _full((1, D)),
                  _full((1, D)), _full((1, D))],
        out_specs=[eb, eb],
        out_shape=[_SDS((NEP, D), _BF16), _SDS((NEP, D), _F32)],
    )(a, b, e, we, b0.reshape(1, D), w1, b1.reshape(1, D),
      w2, b2.reshape(1, D), g.reshape(1, D), be.reshape(1, D))


def _node_step_body(a, v, u1a, u1b, b0, w1, b1, w2, b2, g, be, wd, ws,
                    vo, po, qo):
    agg = a[0] + a[1]
    h = _dot(agg, u1a[...]) + _dot(v[...], u1b[...]) + b0[...]
    h = jnp.maximum(h, 0)
    h = jnp.maximum(_dot(h, w1[...]) + b1[...], 0)
    u = _ln(_dot(h, w2[...]) + b2[...], g[...], be[...])
    vn = v[...] + u
    vo[...] = vn
    po[...] = _dot(vn, wd[...])
    qo[...] = _dot(vn, ws[...])


def _node_step_last_body(a, v, u1a, u1b, b0, w1, b1, w2, b2, g, be, vo):
    agg = a[0] + a[1]
    h = _dot(agg, u1a[...]) + _dot(v[...], u1b[...]) + b0[...]
    h = jnp.maximum(h, 0)
    h = jnp.maximum(_dot(h, w1[...]) + b1[...], 0)
    u = _ln(_dot(h, w2[...]) + b2[...], g[...], be[...])
    vo[...] = v[...] + u


def _node_step(agg2, v, u1a, u1b, b0, w1, b1, w2, b2, g, be, wd=None, ws=None):
    grid = NN // NBLK
    nb = pl.BlockSpec((NBLK, D), lambda i: (i, 0))
    a2 = pl.BlockSpec((2, NBLK, D), lambda i: (0, i, 0))
    wspecs = [_full((D, D)), _full((D, D)), _full((1, D)),
              _full((D, D)), _full((1, D)),
              _full((D, D)), _full((1, D)),
              _full((1, D)), _full((1, D))]
    args = (agg2, v, u1a, u1b, b0.reshape(1, D), w1, b1.reshape(1, D),
            w2, b2.reshape(1, D), g.reshape(1, D), be.reshape(1, D))
    if wd is None:
        return pl.pallas_call(
            _node_step_last_body,
            grid=(grid,),
            in_specs=[a2, nb] + wspecs,
            out_specs=[nb],
            out_shape=[_SDS((NN, D), _F32)],
        )(*args)[0]
    return pl.pallas_call(
        _node_step_body,
        grid=(grid,),
        in_specs=[a2, nb] + wspecs + [_full((D, D)), _full((D, D))],
        out_specs=[nb, nb, nb],
        out_shape=[_SDS((NN, D), _F32), _SDS((NNP, D), _F32),
                   _SDS((NNP, D), _F32)],
    )(*args, wd, ws)


def _dec_body(v, w0, b0, w1, b1, w2, b2, o):
    h = jnp.maximum(_dot(v[...], w0[...]) + b0[...], 0)
    h = jnp.maximum(_dot(h, w1[...]) + b1[...], 0)
    o[...] = _dot(h, w2[...]) + b2[...]


def _decode(v, ps):
    grid = NN // NBLK
    nb = pl.BlockSpec((NBLK, D), lambda i: (i, 0))
    w2p = jnp.pad(ps[2]['w'], ((0, 0), (0, D - ps[2]['w'].shape[1])))
    b2p = jnp.pad(ps[2]['b'], (0, D - ps[2]['b'].shape[0])).reshape(1, D)
    out = pl.pallas_call(
        _dec_body,
        grid=(grid,),
        in_specs=[nb, _full((D, D)), _full((1, D)),
                  _full((D, D)), _full((1, D)),
                  _full((D, D)), _full((1, D))],
        out_specs=[nb],
        out_shape=[_SDS((NN, D), _F32)],
    )(v, ps[0]['w'], ps[0]['b'].reshape(1, D),
      ps[1]['w'], ps[1]['b'].reshape(1, D), w2p, b2p)[0]
    return out[:, :2]


def _sc_mesh():
    return plsc.VectorSubcoreMesh(core_axis_name="c", subcore_axis_name="s")


def _gather_rows(table, idx2):
    @functools.partial(
        pl.kernel,
        out_type=_SDS((NEP, D), table.dtype),
        mesh=_sc_mesh(),
    )
    def k(t_hbm, i_hbm, o_hbm):
        def body(i_v, o_v):
            pltpu.sync_copy(t_hbm.at[i_v.at[0]], o_v)

        pltpu.emit_pipeline(
            body,
            grid=(NEP // GW,),
            in_specs=[pl.BlockSpec((1, GW), lambda i: (0, i))],
            out_specs=[pl.BlockSpec((GW, D), lambda i: (i, 0))],
            core_axis_name=("c", "s"),
            dimension_semantics=(pltpu.PARALLEL,),
        )(i_hbm, o_hbm)

    return k(table, idx2)


def _gather_pq(p, q, dst2, src2):
    return _gather_rows(p, dst2), _gather_rows(q, src2)


def _scatter_add(m, dst2):
    rows_per_tile = NNP // 16
    zr = 40

    @functools.partial(
        pl.kernel,
        out_type=_SDS((2, NNP, D), _F32),
        mesh=_sc_mesh(),
        scratch_types=[pltpu.VMEM_SHARED((NNP, D), _F32),
                       pltpu.VMEM((zr, D), _F32)],
    )
    def k(m_hbm, d_hbm, out_hbm, acc, zbuf):
        cid = lax.axis_index("c")
        sid = lax.axis_index("s")

        @pl.loop(0, zr)
        def _zrow(r):
            @pl.loop(0, D, step=16)
            def _zcol(c):
                zbuf.at[pl.ds(r, 1), pl.ds(c, 16)][...] = (
                    jnp.zeros((1, 16), _F32))

        base = sid * rows_per_tile

        @pl.loop(0, rows_per_tile, step=zr)
        def _zacc(r):
            pltpu.sync_copy(zbuf, acc.at[pl.ds(base + r, zr)])

        plsc.subcore_barrier()

        def body(m_v, d_v):
            pltpu.sync_copy(m_v, acc.at[d_v.at[0]], add=True)

        pltpu.emit_pipeline(
            body,
            grid=(NEP // GW,),
            in_specs=[pl.BlockSpec((GW, D), lambda i: (i, 0)),
                      pl.BlockSpec((1, GW), lambda i: (0, i))],
            out_specs=[],
            core_axis_name=("c", "s"),
            dimension_semantics=(pltpu.PARALLEL,),
        )(m_hbm, d_hbm)

        plsc.subcore_barrier()
        pltpu.sync_copy(acc.at[pl.ds(base, rows_per_tile)],
                        out_hbm.at[cid].at[pl.ds(base, rows_per_tile)])

    return k(m, dst2)


def kernel(x, edge_index, edge_features, params):
    src = edge_index[0]
    dst = edge_index[1]
    pad = NEP - NE
    dst2 = jnp.pad(dst, (0, pad), constant_values=DUMMY).reshape(1, NEP)
    src2 = jnp.pad(src, (0, pad), constant_values=DUMMY).reshape(1, NEP)
    efp = jnp.pad(edge_features, ((0, pad), (0, 0)))

    proc = params['proc']
    w0e = proc[0]['edge_mlp'][0]['w']
    v, p, q = _enc_node(x, params['enc_node'],
                        params['enc_node_ln']['g'], params['enc_node_ln']['b'],
                        w0e[:D], w0e[D:2 * D])
    e = _enc_edge(efp, params['enc_edge'],
                  params['enc_edge_ln']['g'], params['enc_edge_ln']['b'])

    for s, ps in enumerate(proc):
        em = ps['edge_mlp']
        w0 = em[0]['w']
        a, b = _gather_pq(p, q, dst2, src2)
        e, m = _edge_step(a, b, e, w0[2 * D:], em[0]['b'],
                          em[1]['w'], em[1]['b'], em[2]['w'], em[2]['b'],
                          ps['edge_ln']['g'], ps['edge_ln']['b'])
        agg2 = _scatter_add(m, dst2)
        nm = ps['node_mlp']
        u1 = nm[0]['w']
        if s + 1 < len(proc):
            w0n = proc[s + 1]['edge_mlp'][0]['w']
            v, p, q = _node_step(agg2, v, u1[:D], u1[D:], nm[0]['b'],
                                 nm[1]['w'], nm[1]['b'], nm[2]['w'],
                                 nm[2]['b'], ps['node_ln']['g'],
                                 ps['node_ln']['b'], w0n[:D], w0n[D:2 * D])
        else:
            v = _node_step(agg2, v, u1[:D], u1[D:], nm[0]['b'],
                           nm[1]['w'], nm[1]['b'], nm[2]['w'], nm[2]['b'],
                           ps['node_ln']['g'], ps['node_ln']['b'])

    return _decode(v, params['dec'])


# speedup vs baseline: 6.4735x; 1.1194x over previous
import functools

import jax
import jax.numpy as jnp
from jax import lax
from jax.experimental import pallas as pl
from jax.experimental.pallas import tpu as pltpu
from jax.experimental.pallas import tpu_sc as plsc

NN = 10000
NE = 320000
D = 128

GW = 128
NW = 32
NEP = 327680
NNP = 10240
DUMMY = NN

EBLK = 4096
NBLK = 1000

_SDS = jax.ShapeDtypeStruct
_F32 = jnp.float32
_BF16 = jnp.bfloat16


def _ln(h, g, b):
    mu = jnp.mean(h, axis=-1, keepdims=True)
    var = jnp.mean((h - mu) ** 2, axis=-1, keepdims=True)
    return (h - mu) * lax.rsqrt(var + 1e-5) * g + b


def _dot(a, b):
    return jnp.dot(a, b, preferred_element_type=_F32,
                   precision=lax.Precision.HIGHEST)


def _full(shape):
    return pl.BlockSpec(shape, lambda i: (0,) * len(shape))


def _enc_node_body(x, w0, b0, w1, b1, w2, b2, g, be, wd, ws,
                   v_o, p_o, q_o):
    h = jnp.maximum(_dot(x[...], w0[...]) + b0[...], 0)
    h = jnp.maximum(_dot(h, w1[...]) + b1[...], 0)
    v = _ln(_dot(h, w2[...]) + b2[...], g[...], be[...])
    v_o[...] = v
    p_o[...] = _dot(v, wd[...])
    q_o[...] = _dot(v, ws[...])


def _enc_node(x, ps, g, be, wd, ws):
    grid = NN // NBLK
    return pl.pallas_call(
        _enc_node_body,
        grid=(grid,),
        in_specs=[pl.BlockSpec((NBLK, 30), lambda i: (i, 0)),
                  _full((30, D)), _full((1, D)),
                  _full((D, D)), _full((1, D)),
                  _full((D, D)), _full((1, D)),
                  _full((1, D)), _full((1, D)),
                  _full((D, D)), _full((D, D))],
        out_specs=[pl.BlockSpec((NBLK, D), lambda i: (i, 0)),
                   pl.BlockSpec((NBLK, D), lambda i: (i, 0)),
                   pl.BlockSpec((NBLK, D), lambda i: (i, 0))],
        out_shape=[_SDS((NN, D), _F32), _SDS((NNP, D), _F32),
                   _SDS((NNP, D), _F32)],
    )(x, ps[0]['w'], ps[0]['b'].reshape(1, D),
      ps[1]['w'], ps[1]['b'].reshape(1, D),
      ps[2]['w'], ps[2]['b'].reshape(1, D),
      g.reshape(1, D), be.reshape(1, D), wd, ws)


def _enc_edge_body(x, w0, b0, w1, b1, w2, b2, g, be, e_o):
    h = jnp.maximum(_dot(x[...], w0[...]) + b0[...], 0)
    h = jnp.maximum(_dot(h, w1[...]) + b1[...], 0)
    e_o[...] = _ln(_dot(h, w2[...]) + b2[...], g[...], be[...])


def _enc_edge(ef, ps, g, be):
    grid = NEP // EBLK
    return pl.pallas_call(
        _enc_edge_body,
        grid=(grid,),
        in_specs=[pl.BlockSpec((EBLK, 3), lambda i: (i, 0)),
                  _full((3, D)), _full((1, D)),
                  _full((D, D)), _full((1, D)),
                  _full((D, D)), _full((1, D)),
                  _full((1, D)), _full((1, D))],
        out_specs=[pl.BlockSpec((EBLK, D), lambda i: (i, 0))],
        out_shape=[_SDS((NEP, D), _F32)],
    )(ef, ps[0]['w'], ps[0]['b'].reshape(1, D),
      ps[1]['w'], ps[1]['b'].reshape(1, D),
      ps[2]['w'], ps[2]['b'].reshape(1, D),
      g.reshape(1, D), be.reshape(1, D))[0]


def _edge_step_body(a, b, e, we, b0, w1, b1, w2, b2, g, be,
                    eo, mo):
    ev = e[...]
    h = (a[...].astype(_F32) + b[...].astype(_F32)
         + _dot(ev, we[...]) + b0[...])
    h = jnp.maximum(h, 0)
    h = jnp.maximum(_dot(h, w1[...]) + b1[...], 0)
    m = _ln(_dot(h, w2[...]) + b2[...], g[...], be[...])
    mo[...] = m
    eo[...] = ev + m


def _edge_step(a, b, e, we, b0, w1, b1, w2, b2, g, be):
    grid = NEP // EBLK
    eb = pl.BlockSpec((EBLK, D), lambda i: (i, 0))
    return pl.pallas_call(
        _edge_step_body,
        grid=(grid,),
        in_specs=[eb, eb, eb,
                  _full((D, D)), _full((1, D)),
                  _full((D, D)), _full((1, D)),
                  _full((D, D)), _full((1, D)),
                  _full((1, D)), _full((1, D))],
        out_specs=[eb, eb],
        out_shape=[_SDS((NEP, D), _F32), _SDS((NEP, D), _F32)],
    )(a, b, e, we, b0.reshape(1, D), w1, b1.reshape(1, D),
      w2, b2.reshape(1, D), g.reshape(1, D), be.reshape(1, D))


def _node_step_body(a, v, u1a, u1b, b0, w1, b1, w2, b2, g, be, wd, ws,
                    vo, po, qo):
    agg = a[0] + a[1]
    h = _dot(agg, u1a[...]) + _dot(v[...], u1b[...]) + b0[...]
    h = jnp.maximum(h, 0)
    h = jnp.maximum(_dot(h, w1[...]) + b1[...], 0)
    u = _ln(_dot(h, w2[...]) + b2[...], g[...], be[...])
    vn = v[...] + u
    vo[...] = vn
    po[...] = _dot(vn, wd[...])
    qo[...] = _dot(vn, ws[...])


def _node_step_last_body(a, v, u1a, u1b, b0, w1, b1, w2, b2, g, be, vo):
    agg = a[0] + a[1]
    h = _dot(agg, u1a[...]) + _dot(v[...], u1b[...]) + b0[...]
    h = jnp.maximum(h, 0)
    h = jnp.maximum(_dot(h, w1[...]) + b1[...], 0)
    u = _ln(_dot(h, w2[...]) + b2[...], g[...], be[...])
    vo[...] = v[...] + u


def _node_step(agg2, v, u1a, u1b, b0, w1, b1, w2, b2, g, be, wd=None, ws=None):
    grid = NN // NBLK
    nb = pl.BlockSpec((NBLK, D), lambda i: (i, 0))
    a2 = pl.BlockSpec((2, NBLK, D), lambda i: (0, i, 0))
    wspecs = [_full((D, D)), _full((D, D)), _full((1, D)),
              _full((D, D)), _full((1, D)),
              _full((D, D)), _full((1, D)),
              _full((1, D)), _full((1, D))]
    args = (agg2, v, u1a, u1b, b0.reshape(1, D), w1, b1.reshape(1, D),
            w2, b2.reshape(1, D), g.reshape(1, D), be.reshape(1, D))
    if wd is None:
        return pl.pallas_call(
            _node_step_last_body,
            grid=(grid,),
            in_specs=[a2, nb] + wspecs,
            out_specs=[nb],
            out_shape=[_SDS((NN, D), _F32)],
        )(*args)[0]
    return pl.pallas_call(
        _node_step_body,
        grid=(grid,),
        in_specs=[a2, nb] + wspecs + [_full((D, D)), _full((D, D))],
        out_specs=[nb, nb, nb],
        out_shape=[_SDS((NN, D), _F32), _SDS((NNP, D), _F32),
                   _SDS((NNP, D), _F32)],
    )(*args, wd, ws)


def _dec_body(v, w0, b0, w1, b1, w2, b2, o):
    h = jnp.maximum(_dot(v[...], w0[...]) + b0[...], 0)
    h = jnp.maximum(_dot(h, w1[...]) + b1[...], 0)
    o[...] = _dot(h, w2[...]) + b2[...]


def _decode(v, ps):
    grid = NN // NBLK
    nb = pl.BlockSpec((NBLK, D), lambda i: (i, 0))
    w2p = jnp.pad(ps[2]['w'], ((0, 0), (0, D - ps[2]['w'].shape[1])))
    b2p = jnp.pad(ps[2]['b'], (0, D - ps[2]['b'].shape[0])).reshape(1, D)
    out = pl.pallas_call(
        _dec_body,
        grid=(grid,),
        in_specs=[nb, _full((D, D)), _full((1, D)),
                  _full((D, D)), _full((1, D)),
                  _full((D, D)), _full((1, D))],
        out_specs=[nb],
        out_shape=[_SDS((NN, D), _F32)],
    )(v, ps[0]['w'], ps[0]['b'].reshape(1, D),
      ps[1]['w'], ps[1]['b'].reshape(1, D), w2p, b2p)[0]
    return out[:, :2]


def _sc_mesh():
    return plsc.VectorSubcoreMesh(core_axis_name="c", subcore_axis_name="s")


def _gather_rows(table, idx2):
    @functools.partial(
        pl.kernel,
        out_type=_SDS((NEP, D), table.dtype),
        mesh=_sc_mesh(),
    )
    def k(t_hbm, i_hbm, o_hbm):
        def body(i_v, o_v):
            pltpu.sync_copy(t_hbm.at[i_v.at[0]], o_v)

        pltpu.emit_pipeline(
            body,
            grid=(NEP // GW,),
            in_specs=[pl.BlockSpec((1, GW), lambda i: (0, i))],
            out_specs=[pl.BlockSpec((GW, D), lambda i: (i, 0))],
            core_axis_name=("c", "s"),
            dimension_semantics=(pltpu.PARALLEL,),
        )(i_hbm, o_hbm)

    return k(table, idx2)


def _gather_pq(p, q, dst2, src2):
    return _gather_rows(p, dst2), _gather_rows(q, src2)


def _scatter_add(m, dst2):
    rows_per_tile = NNP // 16
    zr = 40

    @functools.partial(
        pl.kernel,
        out_type=_SDS((2, NNP, D), _F32),
        mesh=_sc_mesh(),
        scratch_types=[pltpu.VMEM_SHARED((NNP, D), _F32),
                       pltpu.VMEM((zr, D), _F32)],
    )
    def k(m_hbm, d_hbm, out_hbm, acc, zbuf):
        cid = lax.axis_index("c")
        sid = lax.axis_index("s")

        @pl.loop(0, zr)
        def _zrow(r):
            @pl.loop(0, D, step=16)
            def _zcol(c):
                zbuf.at[pl.ds(r, 1), pl.ds(c, 16)][...] = (
                    jnp.zeros((1, 16), _F32))

        base = sid * rows_per_tile

        @pl.loop(0, rows_per_tile, step=zr)
        def _zacc(r):
            pltpu.sync_copy(zbuf, acc.at[pl.ds(base + r, zr)])

        plsc.subcore_barrier()

        def body(m_v, d_v):
            pltpu.sync_copy(m_v, acc.at[d_v.at[0]], add=True)

        pltpu.emit_pipeline(
            body,
            grid=(NEP // GW,),
            in_specs=[pl.BlockSpec((GW, D), lambda i: (i, 0)),
                      pl.BlockSpec((1, GW), lambda i: (0, i))],
            out_specs=[],
            core_axis_name=("c", "s"),
            dimension_semantics=(pltpu.PARALLEL,),
        )(m_hbm, d_hbm)

        plsc.subcore_barrier()
        pltpu.sync_copy(acc.at[pl.ds(base, rows_per_tile)],
                        out_hbm.at[cid].at[pl.ds(base, rows_per_tile)])

    return k(m, dst2)


def kernel(x, edge_index, edge_features, params):
    src = edge_index[0]
    dst = edge_index[1]
    pad = NEP - NE
    dst2 = jnp.pad(dst, (0, pad), constant_values=DUMMY).reshape(1, NEP)
    src2 = jnp.pad(src, (0, pad), constant_values=DUMMY).reshape(1, NEP)
    efp = jnp.pad(edge_features, ((0, pad), (0, 0)))

    proc = params['proc']
    w0e = proc[0]['edge_mlp'][0]['w']
    v, p, q = _enc_node(x, params['enc_node'],
                        params['enc_node_ln']['g'], params['enc_node_ln']['b'],
                        w0e[:D], w0e[D:2 * D])
    e = _enc_edge(efp, params['enc_edge'],
                  params['enc_edge_ln']['g'], params['enc_edge_ln']['b'])

    for s, ps in enumerate(proc):
        em = ps['edge_mlp']
        w0 = em[0]['w']
        a, b = _gather_pq(p, q, dst2, src2)
        e, m = _edge_step(a, b, e, w0[2 * D:], em[0]['b'],
                          em[1]['w'], em[1]['b'], em[2]['w'], em[2]['b'],
                          ps['edge_ln']['g'], ps['edge_ln']['b'])
        agg2 = _scatter_add(m, dst2)
        nm = ps['node_mlp']
        u1 = nm[0]['w']
        if s + 1 < len(proc):
            w0n = proc[s + 1]['edge_mlp'][0]['w']
            v, p, q = _node_step(agg2, v, u1[:D], u1[D:], nm[0]['b'],
                                 nm[1]['w'], nm[1]['b'], nm[2]['w'],
                                 nm[2]['b'], ps['node_ln']['g'],
                                 ps['node_ln']['b'], w0n[:D], w0n[D:2 * D])
        else:
            v = _node_step(agg2, v, u1[:D], u1[D:], nm[0]['b'],
                           nm[1]['w'], nm[1]['b'], nm[2]['w'], nm[2]['b'],
                           ps['node_ln']['g'], ps['node_ln']['b'])

    return _decode(v, params['dec'])


# speedup vs baseline: 9.4374x; 1.4578x over previous
import functools

import jax
import jax.numpy as jnp
from jax import lax
from jax.experimental import pallas as pl
from jax.experimental.pallas import tpu as pltpu
from jax.experimental.pallas import tpu_sc as plsc

NN = 10000
NE = 320000
D = 128

GW = 128
NW = 32
NEP = 327680
NNP = 10240
DUMMY = NN

EBLK = 4096
NBLK = 1000

_SDS = jax.ShapeDtypeStruct
_F32 = jnp.float32
_BF16 = jnp.bfloat16


def _ln(h, g, b):
    mu = jnp.mean(h, axis=-1, keepdims=True)
    var = jnp.mean((h - mu) ** 2, axis=-1, keepdims=True)
    return (h - mu) * lax.rsqrt(var + 1e-5) * g + b


def _dot(a, b):
    return jnp.dot(a, b, preferred_element_type=_F32)


def _full(shape):
    return pl.BlockSpec(shape, lambda i: (0,) * len(shape))


def _enc_node_body(x, w0, b0, w1, b1, w2, b2, g, be, wd, ws,
                   v_o, p_o, q_o):
    h = jnp.maximum(_dot(x[...], w0[...]) + b0[...], 0)
    h = jnp.maximum(_dot(h, w1[...]) + b1[...], 0)
    v = _ln(_dot(h, w2[...]) + b2[...], g[...], be[...])
    v_o[...] = v
    p_o[...] = _dot(v, wd[...])
    q_o[...] = _dot(v, ws[...])


def _enc_node(x, ps, g, be, wd, ws):
    grid = NN // NBLK
    return pl.pallas_call(
        _enc_node_body,
        grid=(grid,),
        in_specs=[pl.BlockSpec((NBLK, 30), lambda i: (i, 0)),
                  _full((30, D)), _full((1, D)),
                  _full((D, D)), _full((1, D)),
                  _full((D, D)), _full((1, D)),
                  _full((1, D)), _full((1, D)),
                  _full((D, D)), _full((D, D))],
        out_specs=[pl.BlockSpec((NBLK, D), lambda i: (i, 0)),
                   pl.BlockSpec((NBLK, D), lambda i: (i, 0)),
                   pl.BlockSpec((NBLK, D), lambda i: (i, 0))],
        out_shape=[_SDS((NN, D), _F32), _SDS((NNP, D), _F32),
                   _SDS((NNP, D), _F32)],
    )(x, ps[0]['w'], ps[0]['b'].reshape(1, D),
      ps[1]['w'], ps[1]['b'].reshape(1, D),
      ps[2]['w'], ps[2]['b'].reshape(1, D),
      g.reshape(1, D), be.reshape(1, D), wd, ws)


def _enc_edge_body(x, w0, b0, w1, b1, w2, b2, g, be, e_o):
    h = jnp.maximum(_dot(x[...], w0[...]) + b0[...], 0)
    h = jnp.maximum(_dot(h, w1[...]) + b1[...], 0)
    e_o[...] = _ln(_dot(h, w2[...]) + b2[...], g[...], be[...])


def _enc_edge(ef, ps, g, be):
    grid = NEP // EBLK
    return pl.pallas_call(
        _enc_edge_body,
        grid=(grid,),
        in_specs=[pl.BlockSpec((EBLK, 3), lambda i: (i, 0)),
                  _full((3, D)), _full((1, D)),
                  _full((D, D)), _full((1, D)),
                  _full((D, D)), _full((1, D)),
                  _full((1, D)), _full((1, D))],
        out_specs=[pl.BlockSpec((EBLK, D), lambda i: (i, 0))],
        out_shape=[_SDS((NEP, D), _F32)],
    )(ef, ps[0]['w'], ps[0]['b'].reshape(1, D),
      ps[1]['w'], ps[1]['b'].reshape(1, D),
      ps[2]['w'], ps[2]['b'].reshape(1, D),
      g.reshape(1, D), be.reshape(1, D))[0]


def _edge_step_body(a, b, e, we, b0, w1, b1, w2, b2, g, be, eo):
    ev = e[...]
    h = (a[...].astype(_F32) + b[...].astype(_F32)
         + _dot(ev, we[...]) + b0[...])
    h = jnp.maximum(h, 0)
    h = jnp.maximum(_dot(h, w1[...]) + b1[...], 0)
    m = _ln(_dot(h, w2[...]) + b2[...], g[...], be[...])
    eo[...] = ev + m


def _edge_step(a, b, e, we, b0, w1, b1, w2, b2, g, be):
    grid = NEP // EBLK
    eb = pl.BlockSpec((EBLK, D), lambda i: (i, 0))
    return pl.pallas_call(
        _edge_step_body,
        grid=(grid,),
        in_specs=[eb, eb, eb,
                  _full((D, D)), _full((1, D)),
                  _full((D, D)), _full((1, D)),
                  _full((D, D)), _full((1, D)),
                  _full((1, D)), _full((1, D))],
        out_specs=[eb],
        out_shape=[_SDS((NEP, D), _F32)],
    )(a, b, e, we, b0.reshape(1, D), w1, b1.reshape(1, D),
      w2, b2.reshape(1, D), g.reshape(1, D), be.reshape(1, D))


def _node_step_body(a, ae, v, u1a, u1b, b0, w1, b1, w2, b2, g, be, wd, ws,
                    vo, po, qo, aeo):
    s_new = a[0] + a[1]
    aeo[...] = s_new
    agg = s_new - ae[...]
    h = _dot(agg, u1a[...]) + _dot(v[...], u1b[...]) + b0[...]
    h = jnp.maximum(h, 0)
    h = jnp.maximum(_dot(h, w1[...]) + b1[...], 0)
    u = _ln(_dot(h, w2[...]) + b2[...], g[...], be[...])
    vn = v[...] + u
    vo[...] = vn
    po[...] = _dot(vn, wd[...])
    qo[...] = _dot(vn, ws[...])


def _node_step_last_body(a, ae, v, u1a, u1b, b0, w1, b1, w2, b2, g, be, vo):
    agg = a[0] + a[1] - ae[...]
    h = _dot(agg, u1a[...]) + _dot(v[...], u1b[...]) + b0[...]
    h = jnp.maximum(h, 0)
    h = jnp.maximum(_dot(h, w1[...]) + b1[...], 0)
    u = _ln(_dot(h, w2[...]) + b2[...], g[...], be[...])
    vo[...] = v[...] + u


def _node_step(agg2, agge, v, u1a, u1b, b0, w1, b1, w2, b2, g, be,
               wd=None, ws=None):
    grid = NN // NBLK
    nb = pl.BlockSpec((NBLK, D), lambda i: (i, 0))
    a2 = pl.BlockSpec((2, NBLK, D), lambda i: (0, i, 0))
    wspecs = [_full((D, D)), _full((D, D)), _full((1, D)),
              _full((D, D)), _full((1, D)),
              _full((D, D)), _full((1, D)),
              _full((1, D)), _full((1, D))]
    args = (agg2, agge, v, u1a, u1b, b0.reshape(1, D), w1, b1.reshape(1, D),
            w2, b2.reshape(1, D), g.reshape(1, D), be.reshape(1, D))
    if wd is None:
        return pl.pallas_call(
            _node_step_last_body,
            grid=(grid,),
            in_specs=[a2, nb, nb] + wspecs,
            out_specs=[nb],
            out_shape=[_SDS((NN, D), _F32)],
        )(*args)[0]
    return pl.pallas_call(
        _node_step_body,
        grid=(grid,),
        in_specs=[a2, nb, nb] + wspecs + [_full((D, D)), _full((D, D))],
        out_specs=[nb, nb, nb, nb],
        out_shape=[_SDS((NN, D), _F32), _SDS((NNP, D), _F32),
                   _SDS((NNP, D), _F32), _SDS((NNP, D), _F32)],
    )(*args, wd, ws)


def _dec_body(v, w0, b0, w1, b1, w2, b2, o):
    h = jnp.maximum(_dot(v[...], w0[...]) + b0[...], 0)
    h = jnp.maximum(_dot(h, w1[...]) + b1[...], 0)
    o[...] = _dot(h, w2[...]) + b2[...]


def _decode(v, ps):
    grid = NN // NBLK
    nb = pl.BlockSpec((NBLK, D), lambda i: (i, 0))
    w2p = jnp.pad(ps[2]['w'], ((0, 0), (0, D - ps[2]['w'].shape[1])))
    b2p = jnp.pad(ps[2]['b'], (0, D - ps[2]['b'].shape[0])).reshape(1, D)
    out = pl.pallas_call(
        _dec_body,
        grid=(grid,),
        in_specs=[nb, _full((D, D)), _full((1, D)),
                  _full((D, D)), _full((1, D)),
                  _full((D, D)), _full((1, D))],
        out_specs=[nb],
        out_shape=[_SDS((NN, D), _F32)],
    )(v, ps[0]['w'], ps[0]['b'].reshape(1, D),
      ps[1]['w'], ps[1]['b'].reshape(1, D), w2p, b2p)[0]
    return out[:, :2]


def _sc_mesh():
    return plsc.VectorSubcoreMesh(core_axis_name="c", subcore_axis_name="s")


def _gather_rows(table, idx2):
    @functools.partial(
        pl.kernel,
        out_type=_SDS((NEP, D), table.dtype),
        mesh=_sc_mesh(),
    )
    def k(t_hbm, i_hbm, o_hbm):
        def body(i_v, o_v):
            pltpu.sync_copy(t_hbm.at[i_v.at[0]], o_v)

        pltpu.emit_pipeline(
            body,
            grid=(NEP // GW,),
            in_specs=[pl.BlockSpec((1, GW), lambda i: (0, i))],
            out_specs=[pl.BlockSpec((GW, D), lambda i: (i, 0))],
            core_axis_name=("c", "s"),
            dimension_semantics=(pltpu.PARALLEL,),
        )(i_hbm, o_hbm)

    return k(table, idx2)


def _gather_pq(p, q, dst2, src2):
    return _gather_rows(p, dst2), _gather_rows(q, src2)


def _scatter_add(m, dst2):
    rows_per_tile = NNP // 16
    zr = 40

    @functools.partial(
        pl.kernel,
        out_type=_SDS((2, NNP, D), _F32),
        mesh=_sc_mesh(),
        scratch_types=[pltpu.VMEM_SHARED((NNP, D), _F32),
                       pltpu.VMEM((zr, D), _F32)],
    )
    def k(m_hbm, d_hbm, out_hbm, acc, zbuf):
        cid = lax.axis_index("c")
        sid = lax.axis_index("s")

        @pl.loop(0, zr)
        def _zrow(r):
            @pl.loop(0, D, step=16)
            def _zcol(c):
                zbuf.at[pl.ds(r, 1), pl.ds(c, 16)][...] = (
                    jnp.zeros((1, 16), _F32))

        base = sid * rows_per_tile

        @pl.loop(0, rows_per_tile, step=zr)
        def _zacc(r):
            pltpu.sync_copy(zbuf, acc.at[pl.ds(base + r, zr)])

        plsc.subcore_barrier()

        def body(m_v, d_v):
            pltpu.sync_copy(m_v, acc.at[d_v.at[0]], add=True)

        pltpu.emit_pipeline(
            body,
            grid=(NEP // GW,),
            in_specs=[pl.BlockSpec((GW, D), lambda i: (i, 0)),
                      pl.BlockSpec((1, GW), lambda i: (0, i))],
            out_specs=[],
            core_axis_name=("c", "s"),
            dimension_semantics=(pltpu.PARALLEL,),
        )(m_hbm, d_hbm)

        plsc.subcore_barrier()
        pltpu.sync_copy(acc.at[pl.ds(base, rows_per_tile)],
                        out_hbm.at[cid].at[pl.ds(base, rows_per_tile)])

    return k(m, dst2)


def kernel(x, edge_index, edge_features, params):
    src = edge_index[0]
    dst = edge_index[1]
    pad = NEP - NE
    dst2 = jnp.pad(dst, (0, pad), constant_values=DUMMY).reshape(1, NEP)
    src2 = jnp.pad(src, (0, pad), constant_values=DUMMY).reshape(1, NEP)
    efp = jnp.pad(edge_features, ((0, pad), (0, 0)))

    proc = params['proc']
    w0e = proc[0]['edge_mlp'][0]['w']
    v, p, q = _enc_node(x, params['enc_node'],
                        params['enc_node_ln']['g'], params['enc_node_ln']['b'],
                        w0e[:D], w0e[D:2 * D])
    e = _enc_edge(efp, params['enc_edge'],
                  params['enc_edge_ln']['g'], params['enc_edge_ln']['b'])

    agg2_0 = _scatter_add(e, dst2)
    agge = agg2_0[0] + agg2_0[1]

    for s, ps in enumerate(proc):
        em = ps['edge_mlp']
        w0 = em[0]['w']
        a, b = _gather_pq(p, q, dst2, src2)
        e = _edge_step(a, b, e, w0[2 * D:], em[0]['b'],
                       em[1]['w'], em[1]['b'], em[2]['w'], em[2]['b'],
                       ps['edge_ln']['g'], ps['edge_ln']['b'])[0]
        agg2 = _scatter_add(e, dst2)
        nm = ps['node_mlp']
        u1 = nm[0]['w']
        if s + 1 < len(proc):
            w0n = proc[s + 1]['edge_mlp'][0]['w']
            v, p, q, agge = _node_step(agg2, agge, v, u1[:D], u1[D:],
                                       nm[0]['b'], nm[1]['w'], nm[1]['b'],
                                       nm[2]['w'], nm[2]['b'],
                                       ps['node_ln']['g'],
                                       ps['node_ln']['b'],
                                       w0n[:D], w0n[D:2 * D])
        else:
            v = _node_step(agg2, agge, v, u1[:D], u1[D:], nm[0]['b'],
                           nm[1]['w'], nm[1]['b'], nm[2]['w'], nm[2]['b'],
                           ps['node_ln']['g'], ps['node_ln']['b'])

    return _decode(v, params['dec'])


# speedup vs baseline: 9.4470x; 1.0010x over previous
import functools

import jax
import jax.numpy as jnp
from jax import lax
from jax.experimental import pallas as pl
from jax.experimental.pallas import tpu as pltpu
from jax.experimental.pallas import tpu_sc as plsc

NN = 10000
NE = 320000
D = 128

GW = 128
NW = 32
NEP = 327680
NNP = 10240
DUMMY = NN

EBLK = 4096
NBLK = 1000

_SDS = jax.ShapeDtypeStruct
_F32 = jnp.float32
_BF16 = jnp.bfloat16


def _ln(h, g, b):
    mu = jnp.mean(h, axis=-1, keepdims=True)
    var = jnp.mean((h - mu) ** 2, axis=-1, keepdims=True)
    return (h - mu) * lax.rsqrt(var + 1e-5) * g + b


def _dot(a, b):
    return jnp.dot(a, b, preferred_element_type=_F32)


_dotx = _dot


def _full(shape):
    return pl.BlockSpec(shape, lambda i: (0,) * len(shape))


def _enc_node_body(x, w0, b0, w1, b1, w2, b2, g, be, wd, ws,
                   v_o, p_o, q_o):
    h = jnp.maximum(_dotx(x[...], w0[...]) + b0[...], 0)
    h = jnp.maximum(_dotx(h, w1[...]) + b1[...], 0)
    v = _ln(_dotx(h, w2[...]) + b2[...], g[...], be[...])
    v_o[...] = v
    p_o[...] = _dotx(v, wd[...])
    q_o[...] = _dotx(v, ws[...])


def _enc_node(x, ps, g, be, wd, ws):
    grid = NN // NBLK
    return pl.pallas_call(
        _enc_node_body,
        grid=(grid,),
        in_specs=[pl.BlockSpec((NBLK, 30), lambda i: (i, 0)),
                  _full((30, D)), _full((1, D)),
                  _full((D, D)), _full((1, D)),
                  _full((D, D)), _full((1, D)),
                  _full((1, D)), _full((1, D)),
                  _full((D, D)), _full((D, D))],
        out_specs=[pl.BlockSpec((NBLK, D), lambda i: (i, 0)),
                   pl.BlockSpec((NBLK, D), lambda i: (i, 0)),
                   pl.BlockSpec((NBLK, D), lambda i: (i, 0))],
        out_shape=[_SDS((NN, D), _F32), _SDS((NNP, D), _F32),
                   _SDS((NNP, D), _F32)],
    )(x, ps[0]['w'], ps[0]['b'].reshape(1, D),
      ps[1]['w'], ps[1]['b'].reshape(1, D),
      ps[2]['w'], ps[2]['b'].reshape(1, D),
      g.reshape(1, D), be.reshape(1, D), wd, ws)


def _enc_edge_body(x, w0, b0, w1, b1, w2, b2, g, be, e_o):
    h = jnp.maximum(_dot(x[...], w0[...]) + b0[...], 0)
    h = jnp.maximum(_dot(h, w1[...]) + b1[...], 0)
    e_o[...] = _ln(_dot(h, w2[...]) + b2[...], g[...], be[...])


def _enc_edge(ef, ps, g, be):
    grid = NEP // EBLK
    return pl.pallas_call(
        _enc_edge_body,
        grid=(grid,),
        in_specs=[pl.BlockSpec((EBLK, 3), lambda i: (i, 0)),
                  _full((3, D)), _full((1, D)),
                  _full((D, D)), _full((1, D)),
                  _full((D, D)), _full((1, D)),
                  _full((1, D)), _full((1, D))],
        out_specs=[pl.BlockSpec((EBLK, D), lambda i: (i, 0))],
        out_shape=[_SDS((NEP, D), _F32)],
    )(ef, ps[0]['w'], ps[0]['b'].reshape(1, D),
      ps[1]['w'], ps[1]['b'].reshape(1, D),
      ps[2]['w'], ps[2]['b'].reshape(1, D),
      g.reshape(1, D), be.reshape(1, D))[0]


def _edge_step_body(a, b, e, we, b0, w1, b1, w2, b2, g, be, eo):
    ev = e[...]
    h = (a[...].astype(_F32) + b[...].astype(_F32)
         + _dot(ev, we[...]) + b0[...])
    h = jnp.maximum(h, 0)
    h = jnp.maximum(_dot(h, w1[...]) + b1[...], 0)
    m = _ln(_dot(h, w2[...]) + b2[...], g[...], be[...])
    eo[...] = ev + m


def _edge_step(a, b, e, we, b0, w1, b1, w2, b2, g, be):
    grid = NEP // EBLK
    eb = pl.BlockSpec((EBLK, D), lambda i: (i, 0))
    return pl.pallas_call(
        _edge_step_body,
        grid=(grid,),
        in_specs=[eb, eb, eb,
                  _full((D, D)), _full((1, D)),
                  _full((D, D)), _full((1, D)),
                  _full((D, D)), _full((1, D)),
                  _full((1, D)), _full((1, D))],
        out_specs=[eb],
        out_shape=[_SDS((NEP, D), _F32)],
    )(a, b, e, we, b0.reshape(1, D), w1, b1.reshape(1, D),
      w2, b2.reshape(1, D), g.reshape(1, D), be.reshape(1, D))


def _node_step_body(a, ae, v, u1a, u1b, b0, w1, b1, w2, b2, g, be, wd, ws,
                    vo, po, qo, aeo):
    s_new = a[0] + a[1]
    aeo[...] = s_new
    agg = s_new - ae[...]
    h = _dotx(agg, u1a[...]) + _dotx(v[...], u1b[...]) + b0[...]
    h = jnp.maximum(h, 0)
    h = jnp.maximum(_dotx(h, w1[...]) + b1[...], 0)
    u = _ln(_dotx(h, w2[...]) + b2[...], g[...], be[...])
    vn = v[...] + u
    vo[...] = vn
    po[...] = _dotx(vn, wd[...])
    qo[...] = _dotx(vn, ws[...])


def _node_step_last_body(a, ae, v, u1a, u1b, b0, w1, b1, w2, b2, g, be, vo):
    agg = a[0] + a[1] - ae[...]
    h = _dotx(agg, u1a[...]) + _dotx(v[...], u1b[...]) + b0[...]
    h = jnp.maximum(h, 0)
    h = jnp.maximum(_dotx(h, w1[...]) + b1[...], 0)
    u = _ln(_dotx(h, w2[...]) + b2[...], g[...], be[...])
    vo[...] = v[...] + u


def _node_step(agg2, agge, v, u1a, u1b, b0, w1, b1, w2, b2, g, be,
               wd=None, ws=None):
    grid = NN // NBLK
    nb = pl.BlockSpec((NBLK, D), lambda i: (i, 0))
    a2 = pl.BlockSpec((2, NBLK, D), lambda i: (0, i, 0))
    wspecs = [_full((D, D)), _full((D, D)), _full((1, D)),
              _full((D, D)), _full((1, D)),
              _full((D, D)), _full((1, D)),
              _full((1, D)), _full((1, D))]
    args = (agg2, agge, v, u1a, u1b, b0.reshape(1, D), w1, b1.reshape(1, D),
            w2, b2.reshape(1, D), g.reshape(1, D), be.reshape(1, D))
    if wd is None:
        return pl.pallas_call(
            _node_step_last_body,
            grid=(grid,),
            in_specs=[a2, nb, nb] + wspecs,
            out_specs=[nb],
            out_shape=[_SDS((NN, D), _F32)],
        )(*args)[0]
    return pl.pallas_call(
        _node_step_body,
        grid=(grid,),
        in_specs=[a2, nb, nb] + wspecs + [_full((D, D)), _full((D, D))],
        out_specs=[nb, nb, nb, nb],
        out_shape=[_SDS((NN, D), _F32), _SDS((NNP, D), _F32),
                   _SDS((NNP, D), _F32), _SDS((NNP, D), _F32)],
    )(*args, wd, ws)


def _dec_body(v, w0, b0, w1, b1, w2, b2, o):
    h = jnp.maximum(_dot(v[...], w0[...]) + b0[...], 0)
    h = jnp.maximum(_dot(h, w1[...]) + b1[...], 0)
    o[...] = _dot(h, w2[...]) + b2[...]


def _decode(v, ps):
    grid = NN // NBLK
    nb = pl.BlockSpec((NBLK, D), lambda i: (i, 0))
    w2p = jnp.pad(ps[2]['w'], ((0, 0), (0, D - ps[2]['w'].shape[1])))
    b2p = jnp.pad(ps[2]['b'], (0, D - ps[2]['b'].shape[0])).reshape(1, D)
    out = pl.pallas_call(
        _dec_body,
        grid=(grid,),
        in_specs=[nb, _full((D, D)), _full((1, D)),
                  _full((D, D)), _full((1, D)),
                  _full((D, D)), _full((1, D))],
        out_specs=[nb],
        out_shape=[_SDS((NN, D), _F32)],
    )(v, ps[0]['w'], ps[0]['b'].reshape(1, D),
      ps[1]['w'], ps[1]['b'].reshape(1, D), w2p, b2p)[0]
    return out[:, :2]


def _sc_mesh():
    return plsc.VectorSubcoreMesh(core_axis_name="c", subcore_axis_name="s")


def _gather_rows(table, idx2):
    @functools.partial(
        pl.kernel,
        out_type=_SDS((NEP, D), table.dtype),
        mesh=_sc_mesh(),
    )
    def k(t_hbm, i_hbm, o_hbm):
        def body(i_v, o_v):
            pltpu.sync_copy(t_hbm.at[i_v.at[0]], o_v)

        pltpu.emit_pipeline(
            body,
            grid=(NEP // GW,),
            in_specs=[pl.BlockSpec((1, GW), lambda i: (0, i))],
            out_specs=[pl.BlockSpec((GW, D), lambda i: (i, 0))],
            core_axis_name=("c", "s"),
            dimension_semantics=(pltpu.PARALLEL,),
        )(i_hbm, o_hbm)

    return k(table, idx2)


def _gather_pq(p, q, dst2, src2):
    return _gather_rows(p, dst2), _gather_rows(q, src2)


def _scatter_add(m, dst2):
    rows_per_tile = NNP // 16
    zr = 40

    @functools.partial(
        pl.kernel,
        out_type=_SDS((2, NNP, D), _F32),
        mesh=_sc_mesh(),
        scratch_types=[pltpu.VMEM_SHARED((NNP, D), _F32),
                       pltpu.VMEM((zr, D), _F32)],
    )
    def k(m_hbm, d_hbm, out_hbm, acc, zbuf):
        cid = lax.axis_index("c")
        sid = lax.axis_index("s")

        @pl.loop(0, zr)
        def _zrow(r):
            @pl.loop(0, D, step=16)
            def _zcol(c):
                zbuf.at[pl.ds(r, 1), pl.ds(c, 16)][...] = (
                    jnp.zeros((1, 16), _F32))

        base = sid * rows_per_tile

        @pl.loop(0, rows_per_tile, step=zr)
        def _zacc(r):
            pltpu.sync_copy(zbuf, acc.at[pl.ds(base + r, zr)])

        plsc.subcore_barrier()

        def body(m_v, d_v):
            pltpu.sync_copy(m_v, acc.at[d_v.at[0]], add=True)

        pltpu.emit_pipeline(
            body,
            grid=(NEP // GW,),
            in_specs=[pl.BlockSpec((GW, D), lambda i: (i, 0)),
                      pl.BlockSpec((1, GW), lambda i: (0, i))],
            out_specs=[],
            core_axis_name=("c", "s"),
            dimension_semantics=(pltpu.PARALLEL,),
        )(m_hbm, d_hbm)

        plsc.subcore_barrier()
        pltpu.sync_copy(acc.at[pl.ds(base, rows_per_tile)],
                        out_hbm.at[cid].at[pl.ds(base, rows_per_tile)])

    return k(m, dst2)


def kernel(x, edge_index, edge_features, params):
    src = edge_index[0]
    dst = edge_index[1]
    pad = NEP - NE
    dst2 = jnp.pad(dst, (0, pad), constant_values=DUMMY).reshape(1, NEP)
    src2 = jnp.pad(src, (0, pad), constant_values=DUMMY).reshape(1, NEP)
    efp = jnp.pad(edge_features, ((0, pad), (0, 0)))

    proc = params['proc']
    w0e = proc[0]['edge_mlp'][0]['w']
    v, p, q = _enc_node(x, params['enc_node'],
                        params['enc_node_ln']['g'], params['enc_node_ln']['b'],
                        w0e[:D], w0e[D:2 * D])
    e = _enc_edge(efp, params['enc_edge'],
                  params['enc_edge_ln']['g'], params['enc_edge_ln']['b'])

    agg2_0 = _scatter_add(e, dst2)
    agge = agg2_0[0] + agg2_0[1]

    for s, ps in enumerate(proc):
        em = ps['edge_mlp']
        w0 = em[0]['w']
        a, b = _gather_pq(p, q, dst2, src2)
        e = _edge_step(a, b, e, w0[2 * D:], em[0]['b'],
                       em[1]['w'], em[1]['b'], em[2]['w'], em[2]['b'],
                       ps['edge_ln']['g'], ps['edge_ln']['b'])[0]
        agg2 = _scatter_add(e, dst2)
        nm = ps['node_mlp']
        u1 = nm[0]['w']
        if s + 1 < len(proc):
            w0n = proc[s + 1]['edge_mlp'][0]['w']
            v, p, q, agge = _node_step(agg2, agge, v, u1[:D], u1[D:],
                                       nm[0]['b'], nm[1]['w'], nm[1]['b'],
                                       nm[2]['w'], nm[2]['b'],
                                       ps['node_ln']['g'],
                                       ps['node_ln']['b'],
                                       w0n[:D], w0n[D:2 * D])
        else:
            v = _node_step(agg2, agge, v, u1[:D], u1[D:], nm[0]['b'],
                           nm[1]['w'], nm[1]['b'], nm[2]['w'], nm[2]['b'],
                           ps['node_ln']['g'], ps['node_ln']['b'])

    return _decode(v, params['dec'])
